# Initial kernel scaffold; baseline (speedup 1.0000x reference)
import jax
import jax.numpy as jnp
from jax.experimental import pallas as pl


def kernel(c1_w, c1_b, c3_w, c3_b, c5_wt, c5_b, f6_wt, f6_b, out_wt, out_b, x):
    raise NotImplementedError("write your pallas kernel here")


# speedup vs baseline: 159.1362x; 159.1362x over previous
import jax
import jax.numpy as jnp
from jax import lax
from jax.experimental import pallas as pl
from jax.experimental.pallas import tpu as pltpu

_VMEM_LIMIT = 64 * 1024 * 1024
_TB = 128


def _round_up(x, m):
    return ((x + m - 1) // m) * m


def _mm(a, w):
    return lax.dot_general(a, w, (((1,), (0,)), ((), ())),
                           preferred_element_type=jnp.float32)


def _band(w, in_w, out_w, stride=1, swap=False):
    co, ci = w.shape[0], w.shape[1]
    ows = stride * jnp.arange(out_w)[None, None, None, :]
    oneh = (jnp.arange(in_w)[None, None, :, None] - ows
            - jnp.arange(stride)[:, None, None, None]
            == jnp.arange(5)[None, :, None, None]).astype(w.dtype)
    b = jnp.einsum('ochk,pkiw->phciwo' if swap else 'ochk,pkiw->phciow',
                   w, oneh)
    return b.reshape(stride, 5, ci * in_w, co * out_w)


def _lenet_kernel(x_ref, w1_ref, b1_ref, w2_ref, b2_ref, w5_ref, b5_ref,
                  f6_ref, b6_ref, wo_ref, bo_ref, o_ref, a1_ref, a2_ref):
    tb = o_ref.shape[0]

    for oc in range(4):
        base = 8 * oc
        acc0 = _mm(x_ref[base:base + 8].reshape(8 * tb, 108), w1_ref[0, 0])
        acc1 = _mm(x_ref[base:base + 8].reshape(8 * tb, 108), w1_ref[1, 0])
        for kh in range(1, 5):
            xs = x_ref[base + kh:base + kh + 8].reshape(8 * tb, 108)
            acc0 = acc0 + _mm(xs, w1_ref[0, kh])
            acc1 = acc1 + _mm(xs, w1_ref[1, kh])
        t = jnp.maximum(acc0, acc1)
        t = t.reshape(4, 2, tb, 96)
        t = jnp.maximum(t[:, 0], t[:, 1])
        a1_ref[4 * oc:4 * oc + 4] = jnp.maximum(t + b1_ref[...], 0.0)

    for oc in range(2):
        base = 6 * oc
        acc0 = _mm(a1_ref[base:base + 6].reshape(6 * tb, 96), w2_ref[0, 0])
        acc1 = _mm(a1_ref[base:base + 6].reshape(6 * tb, 96), w2_ref[1, 0])
        for kh in range(1, 5):
            xs = a1_ref[base + kh:base + kh + 6].reshape(6 * tb, 96)
            acc0 = acc0 + _mm(xs, w2_ref[0, kh])
            acc1 = acc1 + _mm(xs, w2_ref[1, kh])
        t = jnp.maximum(acc0, acc1)
        t = t.reshape(3, 2, tb, 96)
        t = jnp.maximum(t[:, 0], t[:, 1])
        a2_ref[3 * oc:3 * oc + 3] = jnp.maximum(t + b2_ref[...], 0.0)

    acc = _mm(a2_ref[0:2].reshape(2 * tb, 96), w5_ref[0])
    for kh in range(1, 5):
        acc = acc + _mm(a2_ref[kh:kh + 2].reshape(2 * tb, 96), w5_ref[kh])
    a5 = jnp.maximum(acc + b5_ref[...], 0.0).reshape(2, tb, 240)

    h = _mm(a5[0], f6_ref[0]) + _mm(a5[1], f6_ref[1])
    h = jnp.maximum(h + b6_ref[...], 0.0)

    o_ref[...] = _mm(h, wo_ref[...]) + bo_ref[...]


def kernel(c1_w, c1_b, c3_w, c3_b, c5_wt, c5_b, f6_wt, f6_b, out_wt, out_b, x):
    B = x.shape[0]
    f32 = jnp.float32

    w1b = _band(c1_w.reshape(6, 3, 5, 5), 36, 16, stride=2)
    w2b = _band(c3_w.reshape(16, 6, 5, 5), 16, 6, stride=2)
    w5b = _band(c5_wt.T.reshape(120, 16, 5, 5), 6, 2, swap=True)[0]
    b1p = jnp.broadcast_to(c1_b.reshape(6, 1), (6, 16)).reshape(1, 96)
    b2p = jnp.broadcast_to(c3_b.reshape(16, 1), (16, 6)).reshape(1, 96)
    b5t = jnp.concatenate([c5_b.reshape(1, 120)] * 2, axis=1)
    f6c = jnp.stack([jnp.concatenate([f6_wt[0], f6_wt[1]], axis=0),
                     jnp.concatenate([f6_wt[2], f6_wt[3]], axis=0)])
    b6r = f6_b.reshape(1, 84)

    b_pad = _round_up(B, _TB)
    xp = jnp.pad(x, ((0, b_pad - B), (0, 0), (2, 2), (2, 2)))
    xp = jnp.transpose(xp, (2, 0, 1, 3)).reshape(36, b_pad, 108)

    nb = b_pad // _TB
    out = pl.pallas_call(
        _lenet_kernel,
        out_shape=jax.ShapeDtypeStruct((b_pad, 128), f32),
        grid=(nb,),
        in_specs=[
            pl.BlockSpec((36, _TB, 108), lambda i: (0, i, 0)),
            pl.BlockSpec((2, 5, 108, 96), lambda i: (0, 0, 0, 0)),
            pl.BlockSpec((1, 96), lambda i: (0, 0)),
            pl.BlockSpec((2, 5, 96, 96), lambda i: (0, 0, 0, 0)),
            pl.BlockSpec((1, 96), lambda i: (0, 0)),
            pl.BlockSpec((5, 96, 240), lambda i: (0, 0, 0)),
            pl.BlockSpec((1, 240), lambda i: (0, 0)),
            pl.BlockSpec((2, 240, 84), lambda i: (0, 0, 0)),
            pl.BlockSpec((1, 84), lambda i: (0, 0)),
            pl.BlockSpec((84, 128), lambda i: (0, 0)),
            pl.BlockSpec((1, 128), lambda i: (0, 0)),
        ],
        out_specs=pl.BlockSpec((_TB, 128), lambda i: (i, 0)),
        scratch_shapes=[
            pltpu.VMEM((16, _TB, 96), f32),
            pltpu.VMEM((6, _TB, 96), f32),
        ],
        compiler_params=pltpu.CompilerParams(
            dimension_semantics=("parallel",),
            vmem_limit_bytes=_VMEM_LIMIT),
        cost_estimate=pl.CostEstimate(
            flops=2 * b_pad * (32 * 108 * 192 * 5 + 12 * 96 * 192 * 5
                               + 2 * 96 * 240 * 5 + 2 * 240 * 84 + 84 * 128),
            transcendentals=0,
            bytes_accessed=4 * (36 * b_pad * 108 + b_pad * 128)),
    )(xp, w1b, b1p, w2b, b2p, w5b, b5t, f6c, b6r, out_wt, out_b)
    return out[:B, :10]


# speedup vs baseline: 242.8092x; 1.5258x over previous
import jax
import jax.numpy as jnp
from jax import lax
from jax.experimental import pallas as pl
from jax.experimental.pallas import tpu as pltpu

_VMEM_LIMIT = 64 * 1024 * 1024
_TB = 128


def _round_up(x, m):
    return ((x + m - 1) // m) * m


def _mm(a, w):
    return lax.dot_general(a, w, (((1,), (0,)), ((), ())),
                           preferred_element_type=jnp.float32)


def _band(w, in_w, out_w, stride=1, offset=0, swap=False):
    co, ci = w.shape[0], w.shape[1]
    ows = stride * jnp.arange(out_w)[None, None, None, :]
    oneh = (jnp.arange(in_w)[None, None, :, None] - ows + offset
            - jnp.arange(stride)[:, None, None, None]
            == jnp.arange(5)[None, :, None, None]).astype(w.dtype)
    b = jnp.einsum('ochk,pkiw->phciwo' if swap else 'ochk,pkiw->phciow',
                   w, oneh)
    return b.reshape(stride, 5, ci * in_w, co * out_w)


def _lenet_kernel(x_ref, w1_ref, b1_ref, w2_ref, b2_ref, w5_ref, b5_ref,
                  f6_ref, b6_ref, wo_ref, bo_ref, o_ref, xs_ref, a1_ref,
                  a2_ref):
    tb = o_ref.shape[0]

    xs_ref[0:2] = jnp.zeros((2, tb, 96), jnp.float32)
    xs_ref[2:34] = x_ref[...]
    xs_ref[34:36] = jnp.zeros((2, tb, 96), jnp.float32)

    for oc in range(4):
        base = 8 * oc
        acc0 = _mm(xs_ref[base:base + 8].reshape(8 * tb, 96), w1_ref[0, 0])
        acc1 = _mm(xs_ref[base:base + 8].reshape(8 * tb, 96), w1_ref[1, 0])
        for kh in range(1, 5):
            xs = xs_ref[base + kh:base + kh + 8].reshape(8 * tb, 96)
            acc0 = acc0 + _mm(xs, w1_ref[0, kh])
            acc1 = acc1 + _mm(xs, w1_ref[1, kh])
        t = jnp.maximum(acc0, acc1)
        t = t.reshape(4, 2, tb, 96)
        t = jnp.maximum(t[:, 0], t[:, 1])
        a1_ref[4 * oc:4 * oc + 4] = jnp.maximum(t + b1_ref[...], 0.0)

    for oc in range(2):
        base = 6 * oc
        acc0 = _mm(a1_ref[base:base + 6].reshape(6 * tb, 96), w2_ref[0, 0])
        acc1 = _mm(a1_ref[base:base + 6].reshape(6 * tb, 96), w2_ref[1, 0])
        for kh in range(1, 5):
            xs = a1_ref[base + kh:base + kh + 6].reshape(6 * tb, 96)
            acc0 = acc0 + _mm(xs, w2_ref[0, kh])
            acc1 = acc1 + _mm(xs, w2_ref[1, kh])
        t = jnp.maximum(acc0, acc1)
        t = t.reshape(3, 2, tb, 96)
        t = jnp.maximum(t[:, 0], t[:, 1])
        a2_ref[3 * oc:3 * oc + 3] = jnp.maximum(t + b2_ref[...], 0.0)

    acc = _mm(a2_ref[0:2].reshape(2 * tb, 96), w5_ref[0])
    for kh in range(1, 5):
        acc = acc + _mm(a2_ref[kh:kh + 2].reshape(2 * tb, 96), w5_ref[kh])
    a5 = jnp.maximum(acc + b5_ref[...], 0.0).reshape(2, tb, 240)

    h = _mm(a5[0], f6_ref[0]) + _mm(a5[1], f6_ref[1])
    h = jnp.maximum(h + b6_ref[...], 0.0)

    o_ref[...] = _mm(h, wo_ref[...]) + bo_ref[...]


def kernel(c1_w, c1_b, c3_w, c3_b, c5_wt, c5_b, f6_wt, f6_b, out_wt, out_b, x):
    B = x.shape[0]
    f32 = jnp.float32

    w1b = _band(c1_w.reshape(6, 3, 5, 5), 32, 16, stride=2, offset=2)
    w2b = _band(c3_w.reshape(16, 6, 5, 5), 16, 6, stride=2)
    w5b = _band(c5_wt.T.reshape(120, 16, 5, 5), 6, 2, swap=True)[0]
    b1p = jnp.broadcast_to(c1_b.reshape(6, 1), (6, 16)).reshape(1, 96)
    b2p = jnp.broadcast_to(c3_b.reshape(16, 1), (16, 6)).reshape(1, 96)
    b5t = jnp.concatenate([c5_b.reshape(1, 120)] * 2, axis=1)
    f6c = jnp.stack([jnp.concatenate([f6_wt[0], f6_wt[1]], axis=0),
                     jnp.concatenate([f6_wt[2], f6_wt[3]], axis=0)])
    b6r = f6_b.reshape(1, 84)

    b_pad = _round_up(B, _TB)
    xp = jnp.pad(x, ((0, b_pad - B), (0, 0), (0, 0), (0, 0)))
    xp = jnp.transpose(xp, (2, 0, 1, 3)).reshape(32, b_pad, 96)

    nb = b_pad // _TB
    out = pl.pallas_call(
        _lenet_kernel,
        out_shape=jax.ShapeDtypeStruct((b_pad, 128), f32),
        grid=(nb,),
        in_specs=[
            pl.BlockSpec((32, _TB, 96), lambda i: (0, i, 0)),
            pl.BlockSpec((2, 5, 96, 96), lambda i: (0, 0, 0, 0)),
            pl.BlockSpec((1, 96), lambda i: (0, 0)),
            pl.BlockSpec((2, 5, 96, 96), lambda i: (0, 0, 0, 0)),
            pl.BlockSpec((1, 96), lambda i: (0, 0)),
            pl.BlockSpec((5, 96, 240), lambda i: (0, 0, 0)),
            pl.BlockSpec((1, 240), lambda i: (0, 0)),
            pl.BlockSpec((2, 240, 84), lambda i: (0, 0, 0)),
            pl.BlockSpec((1, 84), lambda i: (0, 0)),
            pl.BlockSpec((84, 128), lambda i: (0, 0)),
            pl.BlockSpec((1, 128), lambda i: (0, 0)),
        ],
        out_specs=pl.BlockSpec((_TB, 128), lambda i: (i, 0)),
        scratch_shapes=[
            pltpu.VMEM((36, _TB, 96), f32),
            pltpu.VMEM((16, _TB, 96), f32),
            pltpu.VMEM((6, _TB, 96), f32),
        ],
        compiler_params=pltpu.CompilerParams(
            dimension_semantics=("parallel",),
            vmem_limit_bytes=_VMEM_LIMIT),
        cost_estimate=pl.CostEstimate(
            flops=2 * b_pad * (32 * 96 * 192 * 5 + 12 * 96 * 192 * 5
                               + 2 * 96 * 240 * 5 + 2 * 240 * 84 + 84 * 128),
            transcendentals=0,
            bytes_accessed=4 * (32 * b_pad * 96 + b_pad * 128)),
    )(xp, w1b, b1p, w2b, b2p, w5b, b5t, f6c, b6r, out_wt, out_b)
    return out[:B, :10]
